# baseline (speedup 1.0000x reference)
import functools

import jax
import jax.numpy as jnp
from jax.experimental import pallas as pl
from jax.experimental.pallas import tpu as pltpu

N = 10000
E = 320000
D = 128


def _mm_kernel(a_ref, w_ref, b_ref, o_ref, *, relu):
    acc = jnp.dot(a_ref[...], w_ref[...], preferred_element_type=jnp.float32)
    acc = acc + b_ref[...]
    if relu:
        acc = jnp.maximum(acc, 0.0)
    o_ref[...] = acc


def _mm_bias(a, w, b, relu, block=2000):
    m = a.shape[0]
    pad = (-m) % block
    if pad:
        a = jnp.pad(a, ((0, pad), (0, 0)))
    mp = a.shape[0]
    grid = (mp // block,)
    out = pl.pallas_call(
        functools.partial(_mm_kernel, relu=relu),
        grid=grid,
        in_specs=[
            pl.BlockSpec((block, D), lambda i: (i, 0)),
            pl.BlockSpec((D, D), lambda i: (0, 0)),
            pl.BlockSpec((D,), lambda i: (0,)),
        ],
        out_specs=pl.BlockSpec((block, D), lambda i: (i, 0)),
        out_shape=jax.ShapeDtypeStruct((mp, D), jnp.float32),
    )(a, w, b)
    return out[:m] if pad else out


def _spmm(idx, val, m):
    return jnp.zeros((E, m.shape[1]), m.dtype).at[idx[0]].add(val[:, None] * m[idx[1]])


def kernel(x, edges, xe, Ldo_indices, Ldo_values, Lup_indices, Lup_values, row, col,
           gnn_W1, gnn_b1, gnn_W2, gnn_b2,
           cw_Wdo1, cw_Wup1, cw_Wid1, cw_b1,
           cw_Wdo2, cw_Wup2, cw_Wid2, cw_b2):
    src, dst = edges[0], edges[1]
    deg = jnp.clip(jnp.zeros((N,), x.dtype).at[dst].add(1.0), 1.0, None)
    agg1 = jnp.zeros_like(x).at[dst].add(x[src]) / deg[:, None]
    h = _mm_bias(agg1, gnn_W1, gnn_b1, relu=True)
    agg2 = jnp.zeros_like(x).at[dst].add(h[src]) / deg[:, None]
    h = _mm_bias(agg2, gnn_W2, gnn_b2, relu=True)

    s1do = _spmm(Ldo_indices, Ldo_values, xe)
    s1up = _spmm(Lup_indices, Lup_values, xe)
    zb = jnp.zeros_like(cw_b1)
    he = jnp.maximum(
        _mm_bias(s1do, cw_Wdo1, cw_b1, relu=False)
        + _mm_bias(s1up, cw_Wup1, zb, relu=False)
        + _mm_bias(xe, cw_Wid1, zb, relu=False), 0.0)
    s2do = _spmm(Ldo_indices, Ldo_values, he)
    s2up = _spmm(Lup_indices, Lup_values, he)
    he = jnp.maximum(
        _mm_bias(s2do, cw_Wdo2, cw_b2, relu=False)
        + _mm_bias(s2up, cw_Wup2, zb, relu=False)
        + _mm_bias(he, cw_Wid2, zb, relu=False), 0.0)

    xed = (jnp.zeros((N, D), he.dtype).at[row].add(he)
           + jnp.zeros((N, D), he.dtype).at[col].add(he))
    return jnp.concatenate([h, xed], axis=-1)
